# baseline (speedup 1.0000x reference)
import jax
import jax.numpy as jnp
from jax import lax
from jax.experimental import pallas as pl
from jax.experimental.pallas import tpu as pltpu
from jax.experimental.pallas import tpu_sc as plsc

_N = 10000
_E = 320000
_EP = 327680
_NC = 2
_NS = 16
_NW = _NC * _NS
_BATCH = 128
_NB = _E // _BATCH
_NBP = _EP // _BATCH
_NP = 10240
_ROWS_PER_TILE = _NP // _NS


def _gelu(t):
    return 0.5 * t * (1.0 + lax.erf(t * 0.7071067811865476))


def _sc_mesh():
    return plsc.VectorSubcoreMesh(
        core_axis_name="c", subcore_axis_name="s",
        num_cores=_NC, num_subcores=_NS)


def _sc_gather(width):
    cnt = _NBP // _NW

    def body(table_hbm, idx_hbm, out_hbm, idx_all, rows0, rows1,
             semg0, semg1, semo0, semo1):
        w = lax.axis_index("s") * _NC + lax.axis_index("c")
        base = w * cnt * _BATCH
        pltpu.sync_copy(idx_hbm.at[pl.ds(base, cnt * _BATCH)], idx_all)

        def pair(jj, carry):
            j0 = 2 * jj
            j1 = j0 + 1

            @pl.when(jj > 0)
            def _():
                pltpu.make_async_copy(rows0, out_hbm.at[pl.ds(base, _BATCH)], semo0).wait()
                pltpu.make_async_copy(rows1, out_hbm.at[pl.ds(base, _BATCH)], semo1).wait()

            pltpu.async_copy(table_hbm.at[idx_all.at[pl.ds(j0 * _BATCH, _BATCH)]], rows0, semg0)
            pltpu.async_copy(table_hbm.at[idx_all.at[pl.ds(j1 * _BATCH, _BATCH)]], rows1, semg1)
            pltpu.make_async_copy(table_hbm.at[idx_all.at[pl.ds(j0 * _BATCH, _BATCH)]], rows0, semg0).wait()
            pltpu.async_copy(rows0, out_hbm.at[pl.ds(base + j0 * _BATCH, _BATCH)], semo0)
            pltpu.make_async_copy(table_hbm.at[idx_all.at[pl.ds(j1 * _BATCH, _BATCH)]], rows1, semg1).wait()
            pltpu.async_copy(rows1, out_hbm.at[pl.ds(base + j1 * _BATCH, _BATCH)], semo1)
            return carry

        lax.fori_loop(0, cnt // 2, pair, 0)
        pltpu.make_async_copy(rows0, out_hbm.at[pl.ds(base, _BATCH)], semo0).wait()
        pltpu.make_async_copy(rows1, out_hbm.at[pl.ds(base, _BATCH)], semo1).wait()

    return pl.kernel(
        body,
        out_type=jax.ShapeDtypeStruct((_EP, width), jnp.float32),
        mesh=_sc_mesh(),
        scratch_types=[
            pltpu.VMEM((cnt * _BATCH,), jnp.int32),
            pltpu.VMEM((_BATCH, width), jnp.float32),
            pltpu.VMEM((_BATCH, width), jnp.float32),
            pltpu.SemaphoreType.DMA,
            pltpu.SemaphoreType.DMA,
            pltpu.SemaphoreType.DMA,
            pltpu.SemaphoreType.DMA,
        ],
    )


def _sc_scatter(width):
    def body(vals_hbm, idx_hbm, zeros_hbm, out_hbm, idx_v, rows_v, table_sh):
        c = lax.axis_index("c")
        s = lax.axis_index("s")
        w = s * _NC + c

        pltpu.sync_copy(zeros_hbm, table_sh.at[pl.ds(s * _ROWS_PER_TILE, _ROWS_PER_TILE)])
        plsc.subcore_barrier()

        cnt = (_NB - w + _NW - 1) // _NW

        def step(j, carry):
            base = (w + j * _NW) * _BATCH
            pltpu.sync_copy(idx_hbm.at[pl.ds(base, _BATCH)], idx_v)
            pltpu.sync_copy(vals_hbm.at[pl.ds(base, _BATCH)], rows_v)
            pltpu.sync_copy(rows_v, table_sh.at[idx_v], add=True)
            return carry

        lax.fori_loop(0, cnt, step, 0)
        plsc.subcore_barrier()

        pltpu.sync_copy(table_sh.at[pl.ds(s * _ROWS_PER_TILE, _ROWS_PER_TILE)],
                        out_hbm.at[c, pl.ds(s * _ROWS_PER_TILE, _ROWS_PER_TILE)])

    return pl.kernel(
        body,
        out_type=jax.ShapeDtypeStruct((_NC, _NP, width), jnp.float32),
        mesh=_sc_mesh(),
        scratch_types=[
            pltpu.VMEM((_BATCH,), jnp.int32),
            pltpu.VMEM((_BATCH, width), jnp.float32),
            pltpu.VMEM_SHARED((_NP, width), jnp.float32),
        ],
    )


def _sc_denom():
    groups = _NBP // _NW // 8

    def body(ew_hbm, idx_hbm, znp_hbm, out_hbm,
             ew8_v, idx8_v, dpriv_v, buf_v, acc_v, stage_sh):
        c = lax.axis_index("c")
        s = lax.axis_index("s")
        w = s * _NC + c

        pltpu.sync_copy(znp_hbm, dpriv_v)

        def grp(g, carry):
            rb = w * (_NBP // _NW) + g * 8
            pltpu.sync_copy(ew_hbm.at[pl.ds(rb, 8)], ew8_v)
            pltpu.sync_copy(idx_hbm.at[pl.ds(rb, 8)], idx8_v)
            for r in range(8):
                for k in range(8):
                    idx16 = idx8_v[r, pl.ds(k * 16, 16)]
                    e16 = ew8_v[r, pl.ds(k * 16, 16)]
                    plsc.addupdate_scatter(dpriv_v, [idx16], e16)
            return carry

        lax.fori_loop(0, groups, grp, 0)

        pltpu.sync_copy(dpriv_v, stage_sh.at[s])
        plsc.subcore_barrier()

        pltpu.sync_copy(stage_sh.at[:, pl.ds(s * _ROWS_PER_TILE, _ROWS_PER_TILE)], buf_v)

        def colsum(k, carry):
            a = buf_v[0, pl.ds(k * 16, 16)]
            for r in range(1, _NS):
                a = a + buf_v[r, pl.ds(k * 16, 16)]
            acc_v[pl.ds(k * 16, 16)] = a
            return carry

        lax.fori_loop(0, _ROWS_PER_TILE // 16, colsum, 0)
        pltpu.sync_copy(acc_v, out_hbm.at[c, pl.ds(s * _ROWS_PER_TILE, _ROWS_PER_TILE)])

    return pl.kernel(
        body,
        out_type=jax.ShapeDtypeStruct((_NC, _NP), jnp.float32),
        mesh=_sc_mesh(),
        scratch_types=[
            pltpu.VMEM((8, 128), jnp.float32),
            pltpu.VMEM((8, 128), jnp.int32),
            pltpu.VMEM((_NP,), jnp.float32),
            pltpu.VMEM((_NS, _ROWS_PER_TILE), jnp.float32),
            pltpu.VMEM((_ROWS_PER_TILE,), jnp.float32),
            pltpu.VMEM_SHARED((_NS, _NP), jnp.float32),
        ],
        compiler_params=pltpu.CompilerParams(needs_layout_passes=False),
    )


def _sc_weight():
    groups = _NBP // _NW // 8

    def body(dp_hbm, ew_hbm, idx_hbm, out_hbm, dtot_v, tmp_v, ew8_v, idx8_v, w8_v):
        c = lax.axis_index("c")
        s = lax.axis_index("s")
        w = s * _NC + c

        pltpu.sync_copy(dp_hbm.at[0], dtot_v)
        pltpu.sync_copy(dp_hbm.at[1], tmp_v)

        def addk(k, carry):
            dtot_v[pl.ds(k * 16, 16)] = dtot_v[pl.ds(k * 16, 16)] + tmp_v[pl.ds(k * 16, 16)]
            return carry

        lax.fori_loop(0, _NP // 16, addk, 0)

        def grp(g, carry):
            rb = w * (_NBP // _NW) + g * 8
            pltpu.sync_copy(ew_hbm.at[pl.ds(rb, 8)], ew8_v)
            pltpu.sync_copy(idx_hbm.at[pl.ds(rb, 8)], idx8_v)
            for r in range(8):
                for k in range(8):
                    idx16 = idx8_v[r, pl.ds(k * 16, 16)]
                    e16 = ew8_v[r, pl.ds(k * 16, 16)]
                    d16 = plsc.load_gather(dtot_v, [idx16])
                    w8_v[r, pl.ds(k * 16, 16)] = e16 / jnp.maximum(d16, 1e-12)
            pltpu.sync_copy(w8_v, out_hbm.at[pl.ds(rb, 8)])
            return carry

        lax.fori_loop(0, groups, grp, 0)

    return pl.kernel(
        body,
        out_type=jax.ShapeDtypeStruct((_NBP, 128), jnp.float32),
        mesh=_sc_mesh(),
        scratch_types=[
            pltpu.VMEM((_NP,), jnp.float32),
            pltpu.VMEM((_NP,), jnp.float32),
            pltpu.VMEM((8, 128), jnp.float32),
            pltpu.VMEM((8, 128), jnp.int32),
            pltpu.VMEM((8, 128), jnp.float32),
        ],
        compiler_params=pltpu.CompilerParams(needs_layout_passes=False),
    )


_SC_CACHE = {}


def _gather_rows(table, idx, width):
    key = ("g", width)
    if key not in _SC_CACHE:
        _SC_CACHE[key] = _sc_gather(width)
    return _SC_CACHE[key](table, idx)


def _scatter_rows(vals, idx, zeros_chunk, width):
    key = ("s", width)
    if key not in _SC_CACHE:
        _SC_CACHE[key] = _sc_scatter(width)
    return _SC_CACHE[key](vals, idx, zeros_chunk)


def _denom_part(ew2d, idx2d, znp):
    if "d" not in _SC_CACHE:
        _SC_CACHE["d"] = _sc_denom()
    return _SC_CACHE["d"](ew2d, idx2d, znp)


def _weight_flat(dpart, ew2d, idx2d):
    if "w" not in _SC_CACHE:
        _SC_CACHE["w"] = _sc_weight()
    return _SC_CACHE["w"](dpart, ew2d, idx2d)


def _node_body(x_ref, g_ref, b_ref, w_ref, tsrc_ref, tdst_ref, u_ref):
    xb = x_ref[...]
    mu = jnp.mean(xb, axis=-1, keepdims=True)
    var = jnp.mean((xb - mu) ** 2, axis=-1, keepdims=True)
    h = (xb - mu) / jnp.sqrt(var + 1e-5) * g_ref[...] + b_ref[...]
    p = jnp.dot(h, w_ref[...], preferred_element_type=jnp.float32)
    tsrc_ref[...] = p[:, :256]
    tdst_ref[...] = p[:, 256:384]
    u_ref[...] = p[:, 384:512]


def _edge1_body(gs_ref, gd_ref, emb_ref, wme_ref, wae_ref, bm1_ref, ba1_ref,
                wm2_ref, bm2_ref, wa2_ref, ba2_ref, msg_ref, s_ref, gmax_ref):
    gs = gs_ref[...]
    emb = emb_ref[...]
    pre_m = gs[:, :128] + jnp.dot(emb, wme_ref[...], preferred_element_type=jnp.float32) + bm1_ref[...]
    msg_ref[...] = jnp.dot(_gelu(pre_m), wm2_ref[...], preferred_element_type=jnp.float32) + bm2_ref[...]
    pre_a = gs[:, 128:] + gd_ref[...] + jnp.dot(emb, wae_ref[...], preferred_element_type=jnp.float32) + ba1_ref[...]
    ga = _gelu(pre_a)
    wa2 = wa2_ref[...]
    rows = [
        lax.dot_general(wa2, ga[c * 128:(c + 1) * 128, :],
                        (((1,), (1,)), ((), ())),
                        preferred_element_type=jnp.float32)
        for c in range(ga.shape[0] // 128)
    ]
    s = jnp.concatenate(rows, axis=0) + ba2_ref[...]
    s_ref[...] = s
    bm = jnp.max(s, axis=(0, 1), keepdims=True)

    @pl.when(pl.program_id(0) == 0)
    def _():
        gmax_ref[...] = bm

    @pl.when(pl.program_id(0) != 0)
    def _():
        gmax_ref[...] = jnp.maximum(gmax_ref[...], bm)


def _exp_body(s_ref, gmax_ref, ew_ref):
    ew_ref[...] = jnp.exp(s_ref[...] - gmax_ref[...])


def _edge2_body(msg_ref, w_ref, ident_ref, er_ref):
    wl = w_ref[...]
    ident = ident_ref[...]
    cols = [
        lax.dot_general(ident, wl[c:c + 1, :], (((1,), (1,)), ((), ())),
                        preferred_element_type=jnp.float32)
        for c in range(wl.shape[0])
    ]
    wcol = jnp.concatenate(cols, axis=0)
    er_ref[...] = msg_ref[...] * wcol


def _final_body(x_ref, u_ref, a0_ref, a1_ref, wagg_ref, bself_ref, bagg_ref, o_ref):
    agg = a0_ref[0] + a1_ref[0]
    o_ref[...] = (x_ref[...] + u_ref[...] + bself_ref[...]
                  + jnp.dot(agg, wagg_ref[...], preferred_element_type=jnp.float32)
                  + bagg_ref[...])


def _tc_call(body, grid, in_specs, out_specs, out_shape):
    return pl.pallas_call(body, grid=grid, in_specs=in_specs,
                          out_specs=out_specs, out_shape=out_shape)


def kernel(x, edge_src, edge_dst, edge_emb, gamma1, beta1, W_self, b_self,
           W_m1, b_m1, W_m2, b_m2, W_a1, b_a1, W_a2, b_a2, W_agg, b_agg):
    f32 = jnp.float32
    Wcat = jnp.concatenate([W_m1[:128], W_a1[128:256], W_a1[:128], W_self], axis=1)
    Wme = W_m1[128:]
    Wae = W_a1[256:]
    bm1 = b_m1.reshape(1, 128)
    ba1 = b_a1.reshape(1, 128)
    bm2 = b_m2.reshape(1, 128)
    wa2 = W_a2.reshape(1, 128)
    ba2 = b_a2.reshape(1, 1)
    bself = b_self.reshape(1, 128)
    bagg = b_agg.reshape(1, 128)
    ident = jnp.eye(128, dtype=f32)
    zeros_chunk = jnp.zeros((_ROWS_PER_TILE, 128), f32)
    znp = jnp.zeros((_NP,), f32)
    pad = _EP - _E
    srcp = jnp.pad(edge_src, (0, pad))
    dstg = jnp.pad(edge_dst, (0, pad))
    dsts = jnp.concatenate([edge_dst, jnp.full((pad,), _NP - 1, jnp.int32)])
    embp = jnp.pad(edge_emb, ((0, pad), (0, 0)))
    idx2d = dsts.reshape(_NBP, 128)

    BN = 200
    GN = _N // BN
    tsrc, tdst, u = _tc_call(
        _node_body, (GN,),
        [pl.BlockSpec((BN, 128), lambda i: (i, 0)),
         pl.BlockSpec((128,), lambda i: (0,)),
         pl.BlockSpec((128,), lambda i: (0,)),
         pl.BlockSpec((128, 512), lambda i: (0, 0))],
        [pl.BlockSpec((BN, 256), lambda i: (i, 0)),
         pl.BlockSpec((BN, 128), lambda i: (i, 0)),
         pl.BlockSpec((BN, 128), lambda i: (i, 0))],
        [jax.ShapeDtypeStruct((_N, 256), f32),
         jax.ShapeDtypeStruct((_N, 128), f32),
         jax.ShapeDtypeStruct((_N, 128), f32)],
    )(x, gamma1, beta1, Wcat)

    gs = _gather_rows(tsrc, srcp, 256)
    gd = _gather_rows(tdst, dstg, 128)

    BE = 4096
    GE = _EP // BE
    SB = BE // 128
    msg, s, gmax = _tc_call(
        _edge1_body, (GE,),
        [pl.BlockSpec((BE, 256), lambda i: (i, 0)),
         pl.BlockSpec((BE, 128), lambda i: (i, 0)),
         pl.BlockSpec((BE, 16), lambda i: (i, 0)),
         pl.BlockSpec((16, 128), lambda i: (0, 0)),
         pl.BlockSpec((16, 128), lambda i: (0, 0)),
         pl.BlockSpec((1, 128), lambda i: (0, 0)),
         pl.BlockSpec((1, 128), lambda i: (0, 0)),
         pl.BlockSpec((128, 128), lambda i: (0, 0)),
         pl.BlockSpec((1, 128), lambda i: (0, 0)),
         pl.BlockSpec((1, 128), lambda i: (0, 0)),
         pl.BlockSpec((1, 1), lambda i: (0, 0))],
        [pl.BlockSpec((BE, 128), lambda i: (i, 0)),
         pl.BlockSpec((SB, 128), lambda i: (i, 0)),
         pl.BlockSpec((1, 1), lambda i: (0, 0))],
        [jax.ShapeDtypeStruct((_EP, 128), f32),
         jax.ShapeDtypeStruct((_NBP, 128), f32),
         jax.ShapeDtypeStruct((1, 1), f32)],
    )(gs, gd, embp, Wme, Wae, bm1, ba1, W_m2, bm2, wa2, ba2)

    BX = 320
    ew2d = _tc_call(
        _exp_body, (_NBP // BX,),
        [pl.BlockSpec((BX, 128), lambda i: (i, 0)),
         pl.BlockSpec((1, 1), lambda i: (0, 0))],
        pl.BlockSpec((BX, 128), lambda i: (i, 0)),
        jax.ShapeDtypeStruct((_NBP, 128), f32),
    )(s, gmax)

    dpart = _denom_part(ew2d, idx2d, znp)
    wflat = _weight_flat(dpart, ew2d, idx2d)

    GE2 = (_E + BE - 1) // BE
    er = _tc_call(
        _edge2_body, (GE2,),
        [pl.BlockSpec((BE, 128), lambda i: (i, 0)),
         pl.BlockSpec((SB, 128), lambda i: (i, 0)),
         pl.BlockSpec((128, 128), lambda i: (0, 0))],
        pl.BlockSpec((BE, 128), lambda i: (i, 0)),
        jax.ShapeDtypeStruct((_E, 128), f32),
    )(msg, wflat, ident)

    apart = _scatter_rows(er, edge_dst, zeros_chunk, 128)

    out = _tc_call(
        _final_body, (GN,),
        [pl.BlockSpec((BN, 128), lambda i: (i, 0)),
         pl.BlockSpec((BN, 128), lambda i: (i, 0)),
         pl.BlockSpec((1, BN, 128), lambda i: (0, i, 0)),
         pl.BlockSpec((1, BN, 128), lambda i: (1, i, 0)),
         pl.BlockSpec((128, 128), lambda i: (0, 0)),
         pl.BlockSpec((1, 128), lambda i: (0, 0)),
         pl.BlockSpec((1, 128), lambda i: (0, 0))],
        pl.BlockSpec((BN, 128), lambda i: (i, 0)),
        jax.ShapeDtypeStruct((_N, 128), f32),
    )(x, u, apart, apart, W_agg, bself, bagg)

    return out, er


# speedup vs baseline: 1.2299x; 1.2299x over previous
import jax
import jax.numpy as jnp
from jax import lax
from jax.experimental import pallas as pl
from jax.experimental.pallas import tpu as pltpu
from jax.experimental.pallas import tpu_sc as plsc

_N = 10000
_E = 320000
_EP = 327680
_NC = 2
_NS = 16
_NW = _NC * _NS
_BATCH = 128
_NB = _E // _BATCH
_NBP = _EP // _BATCH
_NP = 10240
_ROWS_PER_TILE = _NP // _NS


def _gelu(t):
    return 0.5 * t * (1.0 + lax.erf(t * 0.7071067811865476))


def _sc_mesh():
    return plsc.VectorSubcoreMesh(
        core_axis_name="c", subcore_axis_name="s",
        num_cores=_NC, num_subcores=_NS)


def _sc_gather_pair():
    cnt = _NBP // _NW

    def body(ts_hbm, td_hbm, sidx_hbm, didx_hbm, gs_hbm, gd_hbm,
             sidx_all, didx_all, s0, s1, d0, d1,
             semS0, semS1, semD0, semD1, semoS0, semoS1, semoD0, semoD1):
        w = lax.axis_index("s") * _NC + lax.axis_index("c")
        base = w * cnt * _BATCH
        pltpu.sync_copy(sidx_hbm.at[pl.ds(base, cnt * _BATCH)], sidx_all)
        pltpu.sync_copy(didx_hbm.at[pl.ds(base, cnt * _BATCH)], didx_all)

        def pair(jj, carry):
            j0 = 2 * jj
            j1 = j0 + 1

            @pl.when(jj > 0)
            def _():
                pltpu.make_async_copy(s0, gs_hbm.at[pl.ds(base, _BATCH)], semoS0).wait()
                pltpu.make_async_copy(d0, gd_hbm.at[pl.ds(base, _BATCH)], semoD0).wait()
                pltpu.make_async_copy(s1, gs_hbm.at[pl.ds(base, _BATCH)], semoS1).wait()
                pltpu.make_async_copy(d1, gd_hbm.at[pl.ds(base, _BATCH)], semoD1).wait()

            pltpu.async_copy(ts_hbm.at[sidx_all.at[pl.ds(j0 * _BATCH, _BATCH)]], s0, semS0)
            pltpu.async_copy(td_hbm.at[didx_all.at[pl.ds(j0 * _BATCH, _BATCH)]], d0, semD0)
            pltpu.async_copy(ts_hbm.at[sidx_all.at[pl.ds(j1 * _BATCH, _BATCH)]], s1, semS1)
            pltpu.async_copy(td_hbm.at[didx_all.at[pl.ds(j1 * _BATCH, _BATCH)]], d1, semD1)
            pltpu.make_async_copy(ts_hbm.at[sidx_all.at[pl.ds(j0 * _BATCH, _BATCH)]], s0, semS0).wait()
            pltpu.async_copy(s0, gs_hbm.at[pl.ds(base + j0 * _BATCH, _BATCH)], semoS0)
            pltpu.make_async_copy(td_hbm.at[didx_all.at[pl.ds(j0 * _BATCH, _BATCH)]], d0, semD0).wait()
            pltpu.async_copy(d0, gd_hbm.at[pl.ds(base + j0 * _BATCH, _BATCH)], semoD0)
            pltpu.make_async_copy(ts_hbm.at[sidx_all.at[pl.ds(j1 * _BATCH, _BATCH)]], s1, semS1).wait()
            pltpu.async_copy(s1, gs_hbm.at[pl.ds(base + j1 * _BATCH, _BATCH)], semoS1)
            pltpu.make_async_copy(td_hbm.at[didx_all.at[pl.ds(j1 * _BATCH, _BATCH)]], d1, semD1).wait()
            pltpu.async_copy(d1, gd_hbm.at[pl.ds(base + j1 * _BATCH, _BATCH)], semoD1)
            return carry

        lax.fori_loop(0, cnt // 2, pair, 0)
        pltpu.make_async_copy(s0, gs_hbm.at[pl.ds(base, _BATCH)], semoS0).wait()
        pltpu.make_async_copy(d0, gd_hbm.at[pl.ds(base, _BATCH)], semoD0).wait()
        pltpu.make_async_copy(s1, gs_hbm.at[pl.ds(base, _BATCH)], semoS1).wait()
        pltpu.make_async_copy(d1, gd_hbm.at[pl.ds(base, _BATCH)], semoD1).wait()

    return pl.kernel(
        body,
        out_type=[jax.ShapeDtypeStruct((_EP, 256), jnp.float32),
                  jax.ShapeDtypeStruct((_EP, 128), jnp.float32)],
        mesh=_sc_mesh(),
        scratch_types=[
            pltpu.VMEM((cnt * _BATCH,), jnp.int32),
            pltpu.VMEM((cnt * _BATCH,), jnp.int32),
            pltpu.VMEM((_BATCH, 256), jnp.float32),
            pltpu.VMEM((_BATCH, 256), jnp.float32),
            pltpu.VMEM((_BATCH, 128), jnp.float32),
            pltpu.VMEM((_BATCH, 128), jnp.float32),
            pltpu.SemaphoreType.DMA,
            pltpu.SemaphoreType.DMA,
            pltpu.SemaphoreType.DMA,
            pltpu.SemaphoreType.DMA,
            pltpu.SemaphoreType.DMA,
            pltpu.SemaphoreType.DMA,
            pltpu.SemaphoreType.DMA,
            pltpu.SemaphoreType.DMA,
        ],
    )


def _sc_scatter(width):
    def body(vals_hbm, idx_hbm, zeros_hbm, out_hbm, idx_v, rows_v, table_sh):
        c = lax.axis_index("c")
        s = lax.axis_index("s")
        w = s * _NC + c

        pltpu.sync_copy(zeros_hbm, table_sh.at[pl.ds(s * _ROWS_PER_TILE, _ROWS_PER_TILE)])
        plsc.subcore_barrier()

        cnt = (_NB - w + _NW - 1) // _NW

        def step(j, carry):
            base = (w + j * _NW) * _BATCH
            pltpu.sync_copy(idx_hbm.at[pl.ds(base, _BATCH)], idx_v)
            pltpu.sync_copy(vals_hbm.at[pl.ds(base, _BATCH)], rows_v)
            pltpu.sync_copy(rows_v, table_sh.at[idx_v], add=True)
            return carry

        lax.fori_loop(0, cnt, step, 0)
        plsc.subcore_barrier()

        pltpu.sync_copy(table_sh.at[pl.ds(s * _ROWS_PER_TILE, _ROWS_PER_TILE)],
                        out_hbm.at[c, pl.ds(s * _ROWS_PER_TILE, _ROWS_PER_TILE)])

    return pl.kernel(
        body,
        out_type=jax.ShapeDtypeStruct((_NC, _NP, width), jnp.float32),
        mesh=_sc_mesh(),
        scratch_types=[
            pltpu.VMEM((_BATCH,), jnp.int32),
            pltpu.VMEM((_BATCH, width), jnp.float32),
            pltpu.VMEM_SHARED((_NP, width), jnp.float32),
        ],
    )


def _sc_denom():
    groups = _NBP // _NW // 8

    def body(ew_hbm, idx_hbm, znp_hbm, out_hbm,
             ew8_v, idx8_v, dpriv_v, buf_v, acc_v, stage_sh):
        c = lax.axis_index("c")
        s = lax.axis_index("s")
        w = s * _NC + c

        pltpu.sync_copy(znp_hbm, dpriv_v)

        def grp(g, carry):
            rb = w * (_NBP // _NW) + g * 8
            pltpu.sync_copy(ew_hbm.at[pl.ds(rb, 8)], ew8_v)
            pltpu.sync_copy(idx_hbm.at[pl.ds(rb, 8)], idx8_v)
            for r in range(8):
                for k in range(8):
                    idx16 = idx8_v[r, pl.ds(k * 16, 16)]
                    e16 = ew8_v[r, pl.ds(k * 16, 16)]
                    plsc.addupdate_scatter(dpriv_v, [idx16], e16)
            return carry

        lax.fori_loop(0, groups, grp, 0)

        pltpu.sync_copy(dpriv_v, stage_sh.at[s])
        plsc.subcore_barrier()

        pltpu.sync_copy(stage_sh.at[:, pl.ds(s * _ROWS_PER_TILE, _ROWS_PER_TILE)], buf_v)

        def colsum(k, carry):
            a = buf_v[0, pl.ds(k * 16, 16)]
            for r in range(1, _NS):
                a = a + buf_v[r, pl.ds(k * 16, 16)]
            acc_v[pl.ds(k * 16, 16)] = a
            return carry

        lax.fori_loop(0, _ROWS_PER_TILE // 16, colsum, 0)
        pltpu.sync_copy(acc_v, out_hbm.at[c, pl.ds(s * _ROWS_PER_TILE, _ROWS_PER_TILE)])

    return pl.kernel(
        body,
        out_type=jax.ShapeDtypeStruct((_NC, _NP), jnp.float32),
        mesh=_sc_mesh(),
        scratch_types=[
            pltpu.VMEM((8, 128), jnp.float32),
            pltpu.VMEM((8, 128), jnp.int32),
            pltpu.VMEM((_NP,), jnp.float32),
            pltpu.VMEM((_NS, _ROWS_PER_TILE), jnp.float32),
            pltpu.VMEM((_ROWS_PER_TILE,), jnp.float32),
            pltpu.VMEM_SHARED((_NS, _NP), jnp.float32),
        ],
        compiler_params=pltpu.CompilerParams(needs_layout_passes=False),
    )


def _sc_weight():
    groups = _NBP // _NW // 8

    def body(dp_hbm, ew_hbm, idx_hbm, out_hbm, dtot_v, tmp_v, ew8_v, idx8_v, w8_v):
        c = lax.axis_index("c")
        s = lax.axis_index("s")
        w = s * _NC + c

        pltpu.sync_copy(dp_hbm.at[0], dtot_v)
        pltpu.sync_copy(dp_hbm.at[1], tmp_v)

        def addk(k, carry):
            dtot_v[pl.ds(k * 16, 16)] = dtot_v[pl.ds(k * 16, 16)] + tmp_v[pl.ds(k * 16, 16)]
            return carry

        lax.fori_loop(0, _NP // 16, addk, 0)

        def grp(g, carry):
            rb = w * (_NBP // _NW) + g * 8
            pltpu.sync_copy(ew_hbm.at[pl.ds(rb, 8)], ew8_v)
            pltpu.sync_copy(idx_hbm.at[pl.ds(rb, 8)], idx8_v)
            for r in range(8):
                for k in range(8):
                    idx16 = idx8_v[r, pl.ds(k * 16, 16)]
                    e16 = ew8_v[r, pl.ds(k * 16, 16)]
                    d16 = plsc.load_gather(dtot_v, [idx16])
                    w8_v[r, pl.ds(k * 16, 16)] = e16 / jnp.maximum(d16, 1e-12)
            pltpu.sync_copy(w8_v, out_hbm.at[pl.ds(rb, 8)])
            return carry

        lax.fori_loop(0, groups, grp, 0)

    return pl.kernel(
        body,
        out_type=jax.ShapeDtypeStruct((_NBP, 128), jnp.float32),
        mesh=_sc_mesh(),
        scratch_types=[
            pltpu.VMEM((_NP,), jnp.float32),
            pltpu.VMEM((_NP,), jnp.float32),
            pltpu.VMEM((8, 128), jnp.float32),
            pltpu.VMEM((8, 128), jnp.int32),
            pltpu.VMEM((8, 128), jnp.float32),
        ],
        compiler_params=pltpu.CompilerParams(needs_layout_passes=False),
    )


_SC_CACHE = {}


def _gather_pair(tsrc, tdst, sidx, didx):
    if "g" not in _SC_CACHE:
        _SC_CACHE["g"] = _sc_gather_pair()
    return _SC_CACHE["g"](tsrc, tdst, sidx, didx)


def _scatter_rows(vals, idx, zeros_chunk, width):
    key = ("s", width)
    if key not in _SC_CACHE:
        _SC_CACHE[key] = _sc_scatter(width)
    return _SC_CACHE[key](vals, idx, zeros_chunk)


def _denom_part(ew2d, idx2d, znp):
    if "d" not in _SC_CACHE:
        _SC_CACHE["d"] = _sc_denom()
    return _SC_CACHE["d"](ew2d, idx2d, znp)


def _weight_flat(dpart, ew2d, idx2d):
    if "w" not in _SC_CACHE:
        _SC_CACHE["w"] = _sc_weight()
    return _SC_CACHE["w"](dpart, ew2d, idx2d)


def _node_body(x_ref, g_ref, b_ref, w_ref, tsrc_ref, tdst_ref, u_ref):
    xb = x_ref[...]
    mu = jnp.mean(xb, axis=-1, keepdims=True)
    var = jnp.mean((xb - mu) ** 2, axis=-1, keepdims=True)
    h = (xb - mu) / jnp.sqrt(var + 1e-5) * g_ref[...] + b_ref[...]
    p = jnp.dot(h, w_ref[...], preferred_element_type=jnp.float32)
    tsrc_ref[...] = p[:, :256]
    tdst_ref[...] = p[:, 256:384]
    u_ref[...] = p[:, 384:512]


def _edge1_body(gs_ref, gd_ref, emb_ref, wme_ref, wae_ref, bm1_ref, ba1_ref,
                wm2_ref, bm2_ref, wa2_ref, ba2_ref, msg_ref, s_ref, gmax_ref):
    gs = gs_ref[...]
    emb = emb_ref[...]
    pre_m = gs[:, :128] + jnp.dot(emb, wme_ref[...], preferred_element_type=jnp.float32) + bm1_ref[...]
    msg_ref[...] = jnp.dot(_gelu(pre_m), wm2_ref[...], preferred_element_type=jnp.float32) + bm2_ref[...]
    pre_a = gs[:, 128:] + gd_ref[...] + jnp.dot(emb, wae_ref[...], preferred_element_type=jnp.float32) + ba1_ref[...]
    ga = _gelu(pre_a)
    wa2 = wa2_ref[...]
    rows = [
        lax.dot_general(wa2, ga[c * 128:(c + 1) * 128, :],
                        (((1,), (1,)), ((), ())),
                        preferred_element_type=jnp.float32)
        for c in range(ga.shape[0] // 128)
    ]
    s = jnp.concatenate(rows, axis=0) + ba2_ref[...]
    s_ref[...] = s
    bm = jnp.max(s, axis=(0, 1), keepdims=True)

    @pl.when(pl.program_id(0) == 0)
    def _():
        gmax_ref[...] = bm

    @pl.when(pl.program_id(0) != 0)
    def _():
        gmax_ref[...] = jnp.maximum(gmax_ref[...], bm)


def _exp_body(s_ref, gmax_ref, ew_ref):
    ew_ref[...] = jnp.exp(s_ref[...] - gmax_ref[...])


def _edge2_body(msg_ref, w_ref, ident_ref, er_ref):
    wl = w_ref[...]
    ident = ident_ref[...]
    cols = [
        lax.dot_general(ident, wl[c:c + 1, :], (((1,), (1,)), ((), ())),
                        preferred_element_type=jnp.float32)
        for c in range(wl.shape[0])
    ]
    wcol = jnp.concatenate(cols, axis=0)
    er_ref[...] = msg_ref[...] * wcol


def _final_body(x_ref, u_ref, a0_ref, a1_ref, wagg_ref, bself_ref, bagg_ref, o_ref):
    agg = a0_ref[0] + a1_ref[0]
    o_ref[...] = (x_ref[...] + u_ref[...] + bself_ref[...]
                  + jnp.dot(agg, wagg_ref[...], preferred_element_type=jnp.float32)
                  + bagg_ref[...])


def _tc_call(body, grid, in_specs, out_specs, out_shape):
    return pl.pallas_call(body, grid=grid, in_specs=in_specs,
                          out_specs=out_specs, out_shape=out_shape)


def kernel(x, edge_src, edge_dst, edge_emb, gamma1, beta1, W_self, b_self,
           W_m1, b_m1, W_m2, b_m2, W_a1, b_a1, W_a2, b_a2, W_agg, b_agg):
    f32 = jnp.float32
    Wcat = jnp.concatenate([W_m1[:128], W_a1[128:256], W_a1[:128], W_self], axis=1)
    Wme = W_m1[128:]
    Wae = W_a1[256:]
    bm1 = b_m1.reshape(1, 128)
    ba1 = b_a1.reshape(1, 128)
    bm2 = b_m2.reshape(1, 128)
    wa2 = W_a2.reshape(1, 128)
    ba2 = b_a2.reshape(1, 1)
    bself = b_self.reshape(1, 128)
    bagg = b_agg.reshape(1, 128)
    ident = jnp.eye(128, dtype=f32)
    zeros_chunk = jnp.zeros((_ROWS_PER_TILE, 128), f32)
    znp = jnp.zeros((_NP,), f32)
    pad = _EP - _E
    srcp = jnp.pad(edge_src, (0, pad))
    dstg = jnp.pad(edge_dst, (0, pad))
    dsts = jnp.concatenate([edge_dst, jnp.full((pad,), _NP - 1, jnp.int32)])
    embp = jnp.pad(edge_emb, ((0, pad), (0, 0)))
    idx2d = dsts.reshape(_NBP, 128)

    BN = 200
    GN = _N // BN
    tsrc, tdst, u = _tc_call(
        _node_body, (GN,),
        [pl.BlockSpec((BN, 128), lambda i: (i, 0)),
         pl.BlockSpec((128,), lambda i: (0,)),
         pl.BlockSpec((128,), lambda i: (0,)),
         pl.BlockSpec((128, 512), lambda i: (0, 0))],
        [pl.BlockSpec((BN, 256), lambda i: (i, 0)),
         pl.BlockSpec((BN, 128), lambda i: (i, 0)),
         pl.BlockSpec((BN, 128), lambda i: (i, 0))],
        [jax.ShapeDtypeStruct((_N, 256), f32),
         jax.ShapeDtypeStruct((_N, 128), f32),
         jax.ShapeDtypeStruct((_N, 128), f32)],
    )(x, gamma1, beta1, Wcat)

    gs, gd = _gather_pair(tsrc, tdst, srcp, dstg)

    BE = 4096
    GE = _EP // BE
    SB = BE // 128
    msg, s, gmax = _tc_call(
        _edge1_body, (GE,),
        [pl.BlockSpec((BE, 256), lambda i: (i, 0)),
         pl.BlockSpec((BE, 128), lambda i: (i, 0)),
         pl.BlockSpec((BE, 16), lambda i: (i, 0)),
         pl.BlockSpec((16, 128), lambda i: (0, 0)),
         pl.BlockSpec((16, 128), lambda i: (0, 0)),
         pl.BlockSpec((1, 128), lambda i: (0, 0)),
         pl.BlockSpec((1, 128), lambda i: (0, 0)),
         pl.BlockSpec((128, 128), lambda i: (0, 0)),
         pl.BlockSpec((1, 128), lambda i: (0, 0)),
         pl.BlockSpec((1, 128), lambda i: (0, 0)),
         pl.BlockSpec((1, 1), lambda i: (0, 0))],
        [pl.BlockSpec((BE, 128), lambda i: (i, 0)),
         pl.BlockSpec((SB, 128), lambda i: (i, 0)),
         pl.BlockSpec((1, 1), lambda i: (0, 0))],
        [jax.ShapeDtypeStruct((_EP, 128), f32),
         jax.ShapeDtypeStruct((_NBP, 128), f32),
         jax.ShapeDtypeStruct((1, 1), f32)],
    )(gs, gd, embp, Wme, Wae, bm1, ba1, W_m2, bm2, wa2, ba2)

    BX = 320
    ew2d = _tc_call(
        _exp_body, (_NBP // BX,),
        [pl.BlockSpec((BX, 128), lambda i: (i, 0)),
         pl.BlockSpec((1, 1), lambda i: (0, 0))],
        pl.BlockSpec((BX, 128), lambda i: (i, 0)),
        jax.ShapeDtypeStruct((_NBP, 128), f32),
    )(s, gmax)

    dpart = _denom_part(ew2d, idx2d, znp)
    wflat = _weight_flat(dpart, ew2d, idx2d)

    GE2 = (_E + BE - 1) // BE
    er = _tc_call(
        _edge2_body, (GE2,),
        [pl.BlockSpec((BE, 128), lambda i: (i, 0)),
         pl.BlockSpec((SB, 128), lambda i: (i, 0)),
         pl.BlockSpec((128, 128), lambda i: (0, 0))],
        pl.BlockSpec((BE, 128), lambda i: (i, 0)),
        jax.ShapeDtypeStruct((_E, 128), f32),
    )(msg, wflat, ident)

    apart = _scatter_rows(er, edge_dst, zeros_chunk, 128)

    out = _tc_call(
        _final_body, (GN,),
        [pl.BlockSpec((BN, 128), lambda i: (i, 0)),
         pl.BlockSpec((BN, 128), lambda i: (i, 0)),
         pl.BlockSpec((1, BN, 128), lambda i: (0, i, 0)),
         pl.BlockSpec((1, BN, 128), lambda i: (1, i, 0)),
         pl.BlockSpec((128, 128), lambda i: (0, 0)),
         pl.BlockSpec((1, 128), lambda i: (0, 0)),
         pl.BlockSpec((1, 128), lambda i: (0, 0))],
        pl.BlockSpec((BN, 128), lambda i: (i, 0)),
        jax.ShapeDtypeStruct((_N, 128), f32),
    )(x, u, apart, apart, W_agg, bself, bagg)

    return out, er


# speedup vs baseline: 1.2820x; 1.0423x over previous
import jax
import jax.numpy as jnp
from jax import lax
from jax.experimental import pallas as pl
from jax.experimental.pallas import tpu as pltpu
from jax.experimental.pallas import tpu_sc as plsc

_N = 10000
_E = 320000
_EP = 327680
_NC = 2
_NS = 16
_NW = _NC * _NS
_BATCH = 128
_NB = _E // _BATCH
_NBP = _EP // _BATCH
_NP = 10240
_ROWS_PER_TILE = _NP // _NS


def _gelu(t):
    return 0.5 * t * (1.0 + lax.erf(t * 0.7071067811865476))


def _sc_mesh():
    return plsc.VectorSubcoreMesh(
        core_axis_name="c", subcore_axis_name="s",
        num_cores=_NC, num_subcores=_NS)


def _sc_gather_pair():
    cnt = _NBP // _NW

    def body(ts_hbm, td_hbm, sidx_hbm, didx_hbm, gs_hbm, gd_hbm,
             sidx_all, didx_all, s0, s1, d0, d1,
             semS0, semS1, semD0, semD1, semoS0, semoS1, semoD0, semoD1):
        w = lax.axis_index("s") * _NC + lax.axis_index("c")
        base = w * cnt * _BATCH
        pltpu.sync_copy(sidx_hbm.at[pl.ds(base, cnt * _BATCH)], sidx_all)
        pltpu.sync_copy(didx_hbm.at[pl.ds(base, cnt * _BATCH)], didx_all)

        def pair(jj, carry):
            j0 = 2 * jj
            j1 = j0 + 1

            @pl.when(jj > 0)
            def _():
                pltpu.make_async_copy(s0, gs_hbm.at[pl.ds(base, _BATCH)], semoS0).wait()
                pltpu.make_async_copy(d0, gd_hbm.at[pl.ds(base, _BATCH)], semoD0).wait()
                pltpu.make_async_copy(s1, gs_hbm.at[pl.ds(base, _BATCH)], semoS1).wait()
                pltpu.make_async_copy(d1, gd_hbm.at[pl.ds(base, _BATCH)], semoD1).wait()

            pltpu.async_copy(ts_hbm.at[sidx_all.at[pl.ds(j0 * _BATCH, _BATCH)]], s0, semS0)
            pltpu.async_copy(td_hbm.at[didx_all.at[pl.ds(j0 * _BATCH, _BATCH)]], d0, semD0)
            pltpu.async_copy(ts_hbm.at[sidx_all.at[pl.ds(j1 * _BATCH, _BATCH)]], s1, semS1)
            pltpu.async_copy(td_hbm.at[didx_all.at[pl.ds(j1 * _BATCH, _BATCH)]], d1, semD1)
            pltpu.make_async_copy(ts_hbm.at[sidx_all.at[pl.ds(j0 * _BATCH, _BATCH)]], s0, semS0).wait()
            pltpu.async_copy(s0, gs_hbm.at[pl.ds(base + j0 * _BATCH, _BATCH)], semoS0)
            pltpu.make_async_copy(td_hbm.at[didx_all.at[pl.ds(j0 * _BATCH, _BATCH)]], d0, semD0).wait()
            pltpu.async_copy(d0, gd_hbm.at[pl.ds(base + j0 * _BATCH, _BATCH)], semoD0)
            pltpu.make_async_copy(ts_hbm.at[sidx_all.at[pl.ds(j1 * _BATCH, _BATCH)]], s1, semS1).wait()
            pltpu.async_copy(s1, gs_hbm.at[pl.ds(base + j1 * _BATCH, _BATCH)], semoS1)
            pltpu.make_async_copy(td_hbm.at[didx_all.at[pl.ds(j1 * _BATCH, _BATCH)]], d1, semD1).wait()
            pltpu.async_copy(d1, gd_hbm.at[pl.ds(base + j1 * _BATCH, _BATCH)], semoD1)
            return carry

        lax.fori_loop(0, cnt // 2, pair, 0)
        pltpu.make_async_copy(s0, gs_hbm.at[pl.ds(base, _BATCH)], semoS0).wait()
        pltpu.make_async_copy(d0, gd_hbm.at[pl.ds(base, _BATCH)], semoD0).wait()
        pltpu.make_async_copy(s1, gs_hbm.at[pl.ds(base, _BATCH)], semoS1).wait()
        pltpu.make_async_copy(d1, gd_hbm.at[pl.ds(base, _BATCH)], semoD1).wait()

    return pl.kernel(
        body,
        out_type=[jax.ShapeDtypeStruct((_EP, 256), jnp.float32),
                  jax.ShapeDtypeStruct((_EP, 128), jnp.float32)],
        mesh=_sc_mesh(),
        scratch_types=[
            pltpu.VMEM((cnt * _BATCH,), jnp.int32),
            pltpu.VMEM((cnt * _BATCH,), jnp.int32),
            pltpu.VMEM((_BATCH, 256), jnp.float32),
            pltpu.VMEM((_BATCH, 256), jnp.float32),
            pltpu.VMEM((_BATCH, 128), jnp.float32),
            pltpu.VMEM((_BATCH, 128), jnp.float32),
            pltpu.SemaphoreType.DMA,
            pltpu.SemaphoreType.DMA,
            pltpu.SemaphoreType.DMA,
            pltpu.SemaphoreType.DMA,
            pltpu.SemaphoreType.DMA,
            pltpu.SemaphoreType.DMA,
            pltpu.SemaphoreType.DMA,
            pltpu.SemaphoreType.DMA,
        ],
    )


def _sc_scatter(width):
    base_cnt = _NB // _NW

    def body(vals_hbm, idx_hbm, zeros_hbm, out_hbm,
             idx0, idx1, rows0, rows1, table_sh,
             semI0, semI1, semV0, semV1, semA0, semA1):
        c = lax.axis_index("c")
        s = lax.axis_index("s")
        w = s * _NC + c
        wbase = (base_cnt * w + jnp.maximum(w - 28, 0)) * _BATCH

        pltpu.sync_copy(zeros_hbm, table_sh.at[pl.ds(s * _ROWS_PER_TILE, _ROWS_PER_TILE)])
        plsc.subcore_barrier()

        def fetch(j, idxb, rowsb, semI, semV):
            pltpu.async_copy(idx_hbm.at[pl.ds(wbase + j * _BATCH, _BATCH)], idxb, semI)
            pltpu.async_copy(vals_hbm.at[pl.ds(wbase + j * _BATCH, _BATCH)], rowsb, semV)

        def pair(jj, carry):
            j0 = 2 * jj
            j1 = j0 + 1

            @pl.when(jj > 0)
            def _():
                pltpu.make_async_copy(rows0, table_sh.at[idx0], semA0).wait()
                pltpu.make_async_copy(rows1, table_sh.at[idx1], semA1).wait()

            fetch(j0, idx0, rows0, semI0, semV0)
            fetch(j1, idx1, rows1, semI1, semV1)
            pltpu.make_async_copy(idx_hbm.at[pl.ds(wbase, _BATCH)], idx0, semI0).wait()
            pltpu.make_async_copy(vals_hbm.at[pl.ds(wbase, _BATCH)], rows0, semV0).wait()
            pltpu.async_copy(rows0, table_sh.at[idx0], semA0, add=True)
            pltpu.make_async_copy(idx_hbm.at[pl.ds(wbase, _BATCH)], idx1, semI1).wait()
            pltpu.make_async_copy(vals_hbm.at[pl.ds(wbase, _BATCH)], rows1, semV1).wait()
            pltpu.async_copy(rows1, table_sh.at[idx1], semA1, add=True)
            return carry

        lax.fori_loop(0, base_cnt // 2, pair, 0)
        pltpu.make_async_copy(rows0, table_sh.at[idx0], semA0).wait()
        pltpu.make_async_copy(rows1, table_sh.at[idx1], semA1).wait()

        @pl.when(w >= 28)
        def _():
            pltpu.sync_copy(idx_hbm.at[pl.ds(wbase + base_cnt * _BATCH, _BATCH)], idx0)
            pltpu.sync_copy(vals_hbm.at[pl.ds(wbase + base_cnt * _BATCH, _BATCH)], rows0)
            pltpu.sync_copy(rows0, table_sh.at[idx0], add=True)

        plsc.subcore_barrier()
        pltpu.sync_copy(table_sh.at[pl.ds(s * _ROWS_PER_TILE, _ROWS_PER_TILE)],
                        out_hbm.at[c, pl.ds(s * _ROWS_PER_TILE, _ROWS_PER_TILE)])

    return pl.kernel(
        body,
        out_type=jax.ShapeDtypeStruct((_NC, _NP, width), jnp.float32),
        mesh=_sc_mesh(),
        scratch_types=[
            pltpu.VMEM((_BATCH,), jnp.int32),
            pltpu.VMEM((_BATCH,), jnp.int32),
            pltpu.VMEM((_BATCH, width), jnp.float32),
            pltpu.VMEM((_BATCH, width), jnp.float32),
            pltpu.VMEM_SHARED((_NP, width), jnp.float32),
            pltpu.SemaphoreType.DMA,
            pltpu.SemaphoreType.DMA,
            pltpu.SemaphoreType.DMA,
            pltpu.SemaphoreType.DMA,
            pltpu.SemaphoreType.DMA,
            pltpu.SemaphoreType.DMA,
        ],
    )


def _sc_denom():
    groups = _NBP // _NW // 8

    def body(ew_hbm, idx_hbm, znp_hbm, out_hbm,
             ew8_v, idx8_v, dpriv_v, buf_v, acc_v, stage_sh):
        c = lax.axis_index("c")
        s = lax.axis_index("s")
        w = s * _NC + c

        pltpu.sync_copy(znp_hbm, dpriv_v)

        def grp(g, carry):
            rb = w * (_NBP // _NW) + g * 8
            pltpu.sync_copy(ew_hbm.at[pl.ds(rb, 8)], ew8_v)
            pltpu.sync_copy(idx_hbm.at[pl.ds(rb, 8)], idx8_v)
            for r in range(8):
                for k in range(8):
                    idx16 = idx8_v[r, pl.ds(k * 16, 16)]
                    e16 = ew8_v[r, pl.ds(k * 16, 16)]
                    plsc.addupdate_scatter(dpriv_v, [idx16], e16)
            return carry

        lax.fori_loop(0, groups, grp, 0)

        pltpu.sync_copy(dpriv_v, stage_sh.at[s])
        plsc.subcore_barrier()

        pltpu.sync_copy(stage_sh.at[:, pl.ds(s * _ROWS_PER_TILE, _ROWS_PER_TILE)], buf_v)

        def colsum(k, carry):
            a = buf_v[0, pl.ds(k * 16, 16)]
            for r in range(1, _NS):
                a = a + buf_v[r, pl.ds(k * 16, 16)]
            acc_v[pl.ds(k * 16, 16)] = a
            return carry

        lax.fori_loop(0, _ROWS_PER_TILE // 16, colsum, 0)
        pltpu.sync_copy(acc_v, out_hbm.at[c, pl.ds(s * _ROWS_PER_TILE, _ROWS_PER_TILE)])

    return pl.kernel(
        body,
        out_type=jax.ShapeDtypeStruct((_NC, _NP), jnp.float32),
        mesh=_sc_mesh(),
        scratch_types=[
            pltpu.VMEM((8, 128), jnp.float32),
            pltpu.VMEM((8, 128), jnp.int32),
            pltpu.VMEM((_NP,), jnp.float32),
            pltpu.VMEM((_NS, _ROWS_PER_TILE), jnp.float32),
            pltpu.VMEM((_ROWS_PER_TILE,), jnp.float32),
            pltpu.VMEM_SHARED((_NS, _NP), jnp.float32),
        ],
        compiler_params=pltpu.CompilerParams(needs_layout_passes=False),
    )


def _sc_weight():
    groups = _NBP // _NW // 8

    def body(dp_hbm, ew_hbm, idx_hbm, out_hbm, dtot_v, tmp_v, ew8_v, idx8_v, w8_v):
        c = lax.axis_index("c")
        s = lax.axis_index("s")
        w = s * _NC + c

        pltpu.sync_copy(dp_hbm.at[0], dtot_v)
        pltpu.sync_copy(dp_hbm.at[1], tmp_v)

        def addk(k, carry):
            dtot_v[pl.ds(k * 16, 16)] = dtot_v[pl.ds(k * 16, 16)] + tmp_v[pl.ds(k * 16, 16)]
            return carry

        lax.fori_loop(0, _NP // 16, addk, 0)

        def grp(g, carry):
            rb = w * (_NBP // _NW) + g * 8
            pltpu.sync_copy(ew_hbm.at[pl.ds(rb, 8)], ew8_v)
            pltpu.sync_copy(idx_hbm.at[pl.ds(rb, 8)], idx8_v)
            for r in range(8):
                for k in range(8):
                    idx16 = idx8_v[r, pl.ds(k * 16, 16)]
                    e16 = ew8_v[r, pl.ds(k * 16, 16)]
                    d16 = plsc.load_gather(dtot_v, [idx16])
                    w8_v[r, pl.ds(k * 16, 16)] = e16 / jnp.maximum(d16, 1e-12)
            pltpu.sync_copy(w8_v, out_hbm.at[pl.ds(rb, 8)])
            return carry

        lax.fori_loop(0, groups, grp, 0)

    return pl.kernel(
        body,
        out_type=jax.ShapeDtypeStruct((_NBP, 128), jnp.float32),
        mesh=_sc_mesh(),
        scratch_types=[
            pltpu.VMEM((_NP,), jnp.float32),
            pltpu.VMEM((_NP,), jnp.float32),
            pltpu.VMEM((8, 128), jnp.float32),
            pltpu.VMEM((8, 128), jnp.int32),
            pltpu.VMEM((8, 128), jnp.float32),
        ],
        compiler_params=pltpu.CompilerParams(needs_layout_passes=False),
    )


_SC_CACHE = {}


def _gather_pair(tsrc, tdst, sidx, didx):
    if "g" not in _SC_CACHE:
        _SC_CACHE["g"] = _sc_gather_pair()
    return _SC_CACHE["g"](tsrc, tdst, sidx, didx)


def _scatter_rows(vals, idx, zeros_chunk, width):
    key = ("s", width)
    if key not in _SC_CACHE:
        _SC_CACHE[key] = _sc_scatter(width)
    return _SC_CACHE[key](vals, idx, zeros_chunk)


def _denom_part(ew2d, idx2d, znp):
    if "d" not in _SC_CACHE:
        _SC_CACHE["d"] = _sc_denom()
    return _SC_CACHE["d"](ew2d, idx2d, znp)


def _weight_flat(dpart, ew2d, idx2d):
    if "w" not in _SC_CACHE:
        _SC_CACHE["w"] = _sc_weight()
    return _SC_CACHE["w"](dpart, ew2d, idx2d)


def _node_body(x_ref, g_ref, b_ref, w_ref, tsrc_ref, tdst_ref, u_ref):
    xb = x_ref[...]
    mu = jnp.mean(xb, axis=-1, keepdims=True)
    var = jnp.mean((xb - mu) ** 2, axis=-1, keepdims=True)
    h = (xb - mu) / jnp.sqrt(var + 1e-5) * g_ref[...] + b_ref[...]
    p = jnp.dot(h, w_ref[...], preferred_element_type=jnp.float32)
    tsrc_ref[...] = p[:, :256]
    tdst_ref[...] = p[:, 256:384]
    u_ref[...] = p[:, 384:512]


def _edge1_body(gs_ref, gd_ref, emb_ref, wme_ref, wae_ref, bm1_ref, ba1_ref,
                wm2_ref, bm2_ref, wa2_ref, ba2_ref, msg_ref, s_ref, gmax_ref):
    gs = gs_ref[...]
    emb = emb_ref[...]
    pre_m = gs[:, :128] + jnp.dot(emb, wme_ref[...], preferred_element_type=jnp.float32) + bm1_ref[...]
    msg_ref[...] = jnp.dot(_gelu(pre_m), wm2_ref[...], preferred_element_type=jnp.float32) + bm2_ref[...]
    pre_a = gs[:, 128:] + gd_ref[...] + jnp.dot(emb, wae_ref[...], preferred_element_type=jnp.float32) + ba1_ref[...]
    ga = _gelu(pre_a)
    wa2 = wa2_ref[...]
    rows = [
        lax.dot_general(wa2, ga[c * 128:(c + 1) * 128, :],
                        (((1,), (1,)), ((), ())),
                        preferred_element_type=jnp.float32)
        for c in range(ga.shape[0] // 128)
    ]
    s = jnp.concatenate(rows, axis=0) + ba2_ref[...]
    s_ref[...] = s
    bm = jnp.max(s, axis=(0, 1), keepdims=True)

    @pl.when(pl.program_id(0) == 0)
    def _():
        gmax_ref[...] = bm

    @pl.when(pl.program_id(0) != 0)
    def _():
        gmax_ref[...] = jnp.maximum(gmax_ref[...], bm)


def _exp_body(s_ref, gmax_ref, ew_ref):
    ew_ref[...] = jnp.exp(s_ref[...] - gmax_ref[...])


def _edge2_body(msg_ref, w_ref, ident_ref, er_ref):
    wl = w_ref[...]
    ident = ident_ref[...]
    cols = [
        lax.dot_general(ident, wl[c:c + 1, :], (((1,), (1,)), ((), ())),
                        preferred_element_type=jnp.float32)
        for c in range(wl.shape[0])
    ]
    wcol = jnp.concatenate(cols, axis=0)
    er_ref[...] = msg_ref[...] * wcol


def _final_body(x_ref, u_ref, a0_ref, a1_ref, wagg_ref, bself_ref, bagg_ref, o_ref):
    agg = a0_ref[0] + a1_ref[0]
    o_ref[...] = (x_ref[...] + u_ref[...] + bself_ref[...]
                  + jnp.dot(agg, wagg_ref[...], preferred_element_type=jnp.float32)
                  + bagg_ref[...])


def _tc_call(body, grid, in_specs, out_specs, out_shape):
    return pl.pallas_call(body, grid=grid, in_specs=in_specs,
                          out_specs=out_specs, out_shape=out_shape)


def kernel(x, edge_src, edge_dst, edge_emb, gamma1, beta1, W_self, b_self,
           W_m1, b_m1, W_m2, b_m2, W_a1, b_a1, W_a2, b_a2, W_agg, b_agg):
    f32 = jnp.float32
    Wcat = jnp.concatenate([W_m1[:128], W_a1[128:256], W_a1[:128], W_self], axis=1)
    Wme = W_m1[128:]
    Wae = W_a1[256:]
    bm1 = b_m1.reshape(1, 128)
    ba1 = b_a1.reshape(1, 128)
    bm2 = b_m2.reshape(1, 128)
    wa2 = W_a2.reshape(1, 128)
    ba2 = b_a2.reshape(1, 1)
    bself = b_self.reshape(1, 128)
    bagg = b_agg.reshape(1, 128)
    ident = jnp.eye(128, dtype=f32)
    zeros_chunk = jnp.zeros((_ROWS_PER_TILE, 128), f32)
    znp = jnp.zeros((_NP,), f32)
    pad = _EP - _E
    srcp = jnp.pad(edge_src, (0, pad))
    dstg = jnp.pad(edge_dst, (0, pad))
    dsts = jnp.concatenate([edge_dst, jnp.full((pad,), _NP - 1, jnp.int32)])
    embp = jnp.pad(edge_emb, ((0, pad), (0, 0)))
    idx2d = dsts.reshape(_NBP, 128)

    BN = 200
    GN = _N // BN
    tsrc, tdst, u = _tc_call(
        _node_body, (GN,),
        [pl.BlockSpec((BN, 128), lambda i: (i, 0)),
         pl.BlockSpec((128,), lambda i: (0,)),
         pl.BlockSpec((128,), lambda i: (0,)),
         pl.BlockSpec((128, 512), lambda i: (0, 0))],
        [pl.BlockSpec((BN, 256), lambda i: (i, 0)),
         pl.BlockSpec((BN, 128), lambda i: (i, 0)),
         pl.BlockSpec((BN, 128), lambda i: (i, 0))],
        [jax.ShapeDtypeStruct((_N, 256), f32),
         jax.ShapeDtypeStruct((_N, 128), f32),
         jax.ShapeDtypeStruct((_N, 128), f32)],
    )(x, gamma1, beta1, Wcat)

    gs, gd = _gather_pair(tsrc, tdst, srcp, dstg)

    BE = 4096
    GE = _EP // BE
    SB = BE // 128
    msg, s, gmax = _tc_call(
        _edge1_body, (GE,),
        [pl.BlockSpec((BE, 256), lambda i: (i, 0)),
         pl.BlockSpec((BE, 128), lambda i: (i, 0)),
         pl.BlockSpec((BE, 16), lambda i: (i, 0)),
         pl.BlockSpec((16, 128), lambda i: (0, 0)),
         pl.BlockSpec((16, 128), lambda i: (0, 0)),
         pl.BlockSpec((1, 128), lambda i: (0, 0)),
         pl.BlockSpec((1, 128), lambda i: (0, 0)),
         pl.BlockSpec((128, 128), lambda i: (0, 0)),
         pl.BlockSpec((1, 128), lambda i: (0, 0)),
         pl.BlockSpec((1, 128), lambda i: (0, 0)),
         pl.BlockSpec((1, 1), lambda i: (0, 0))],
        [pl.BlockSpec((BE, 128), lambda i: (i, 0)),
         pl.BlockSpec((SB, 128), lambda i: (i, 0)),
         pl.BlockSpec((1, 1), lambda i: (0, 0))],
        [jax.ShapeDtypeStruct((_EP, 128), f32),
         jax.ShapeDtypeStruct((_NBP, 128), f32),
         jax.ShapeDtypeStruct((1, 1), f32)],
    )(gs, gd, embp, Wme, Wae, bm1, ba1, W_m2, bm2, wa2, ba2)

    BX = 320
    ew2d = _tc_call(
        _exp_body, (_NBP // BX,),
        [pl.BlockSpec((BX, 128), lambda i: (i, 0)),
         pl.BlockSpec((1, 1), lambda i: (0, 0))],
        pl.BlockSpec((BX, 128), lambda i: (i, 0)),
        jax.ShapeDtypeStruct((_NBP, 128), f32),
    )(s, gmax)

    dpart = _denom_part(ew2d, idx2d, znp)
    wflat = _weight_flat(dpart, ew2d, idx2d)

    GE2 = (_E + BE - 1) // BE
    er = _tc_call(
        _edge2_body, (GE2,),
        [pl.BlockSpec((BE, 128), lambda i: (i, 0)),
         pl.BlockSpec((SB, 128), lambda i: (i, 0)),
         pl.BlockSpec((128, 128), lambda i: (0, 0))],
        pl.BlockSpec((BE, 128), lambda i: (i, 0)),
        jax.ShapeDtypeStruct((_E, 128), f32),
    )(msg, wflat, ident)

    apart = _scatter_rows(er, edge_dst, zeros_chunk, 128)

    out = _tc_call(
        _final_body, (GN,),
        [pl.BlockSpec((BN, 128), lambda i: (i, 0)),
         pl.BlockSpec((BN, 128), lambda i: (i, 0)),
         pl.BlockSpec((1, BN, 128), lambda i: (0, i, 0)),
         pl.BlockSpec((1, BN, 128), lambda i: (1, i, 0)),
         pl.BlockSpec((128, 128), lambda i: (0, 0)),
         pl.BlockSpec((1, 128), lambda i: (0, 0)),
         pl.BlockSpec((1, 128), lambda i: (0, 0))],
        pl.BlockSpec((BN, 128), lambda i: (i, 0)),
        jax.ShapeDtypeStruct((_N, 128), f32),
    )(x, u, apart, apart, W_agg, bself, bagg)

    return out, er
